# baseline (speedup 1.0000x reference)
import functools

import jax
import jax.numpy as jnp
from jax import lax
from jax.experimental import pallas as pl
from jax.experimental.pallas import tpu as pltpu
from jax.experimental.pallas import tpu_sc as plsc

_NC, _NS = 2, 16
_NW = _NC * _NS
_K = 4


@functools.lru_cache(maxsize=None)
def _emb_call(V, D, B0, H):
    rows_per_w = B0 // _NW
    n_halves = rows_per_w // _K
    assert rows_per_w % _K == 0 and n_halves % 2 == 0 and n_halves >= 6
    mesh = plsc.VectorSubcoreMesh(core_axis_name="c", subcore_axis_name="s")

    @functools.partial(
        pl.kernel,
        out_type=jax.ShapeDtypeStruct((B0, H, 2 * D), jnp.float32),
        mesh=mesh,
        scratch_types=[
            pltpu.VMEM((2, _K, H), jnp.int32),
            pltpu.VMEM((2, _K, H, D), jnp.float32),
            pltpu.SemaphoreType.DMA,
            pltpu.SemaphoreType.DMA,
            pltpu.SemaphoreType.DMA,
            pltpu.SemaphoreType.DMA,
            pltpu.SemaphoreType.DMA,
            pltpu.SemaphoreType.DMA,
        ],
        compiler_params=pltpu.CompilerParams(use_tc_tiling_on_sc=False),
    )
    def body(idx_hbm, table_hbm, out_hbm, idx_v, rows_v,
             is0, is1, gs0, gs1, ss0, ss1):
        isem = (is0, is1)
        gsem = (gs0, gs1)
        ssem = (ss0, ss1)
        wid = lax.axis_index("s") * _NC + lax.axis_index("c")
        base = wid * rows_per_w

        def fire_idx_load(h, q):
            pltpu.async_copy(idx_hbm.at[pl.ds(base + h * _K, _K)],
                             idx_v.at[q], isem[q])

        def wait_idx(q):
            pltpu.make_async_copy(idx_hbm.at[pl.ds(0, _K)], idx_v.at[q],
                                  isem[q]).wait()

        def wait_gather(q, k):
            pltpu.make_async_copy(out_hbm.at[0, :, pl.ds(0, D)],
                                  rows_v.at[q, k], gsem[q]).wait()

        def wait_store(q, k):
            pltpu.make_async_copy(rows_v.at[q, k],
                                  out_hbm.at[0, :, pl.ds(0, D)],
                                  ssem[q]).wait()

        _SPLITS = ((0, 104), (104, 96))

        def process_half(h, q, first, prefetch):
            wait_idx(q)
            if not first:
                for k in range(_K):
                    wait_store(q, k)
            for k in range(_K):
                for off, ln in _SPLITS:
                    pltpu.async_copy(
                        table_hbm.at[idx_v.at[q, k, pl.ds(off, ln)]],
                        rows_v.at[q, k, pl.ds(off, ln)], gsem[q])
            if prefetch:
                fire_idx_load(h + 1, 1 - q)
            for k in range(_K):
                wait_gather(q, k)
            for k in range(_K):
                pltpu.async_copy(
                    rows_v.at[q, k],
                    out_hbm.at[base + h * _K + k, :, pl.ds(0, D)], ssem[q])

        fire_idx_load(0, 0)
        process_half(0, 0, first=True, prefetch=True)
        process_half(1, 1, first=True, prefetch=True)

        @pl.loop(2, n_halves - 2, step=2)
        def _steady(Hh):
            process_half(Hh, 0, first=False, prefetch=True)
            process_half(Hh + 1, 1, first=False, prefetch=True)

        process_half(n_halves - 2, 0, first=False, prefetch=True)
        process_half(n_halves - 1, 1, first=False, prefetch=False)

        for q in (0, 1):
            for k in range(_K):
                wait_store(q, k)

    return body


def kernel(x, table):
    B0, H = x.shape
    V, D = table.shape
    out = _emb_call(V, D, B0, H)(x.astype(jnp.int32), table)
    return out[:, :, :D]


# speedup vs baseline: 1.0050x; 1.0050x over previous
import functools

import jax
import jax.numpy as jnp
from jax import lax
from jax.experimental import pallas as pl
from jax.experimental.pallas import tpu as pltpu
from jax.experimental.pallas import tpu_sc as plsc

_NC, _NS = 2, 16
_NW = _NC * _NS
_K = 4


@functools.lru_cache(maxsize=None)
def _emb_call(V, D, B0, H):
    rows_per_w = B0 // _NW
    n_halves = rows_per_w // _K
    assert rows_per_w % _K == 0 and n_halves % 2 == 0 and n_halves >= 6
    mesh = plsc.VectorSubcoreMesh(core_axis_name="c", subcore_axis_name="s")

    @functools.partial(
        pl.kernel,
        out_type=jax.ShapeDtypeStruct((B0, H, 2 * D), jnp.float32),
        mesh=mesh,
        scratch_types=[
            pltpu.VMEM((2, _K, H), jnp.int32),
            pltpu.VMEM((2, _K, H, D), jnp.float32),
            pltpu.SemaphoreType.DMA,
            pltpu.SemaphoreType.DMA,
            pltpu.SemaphoreType.DMA,
            pltpu.SemaphoreType.DMA,
            pltpu.SemaphoreType.DMA,
            pltpu.SemaphoreType.DMA,
        ],
        compiler_params=pltpu.CompilerParams(use_tc_tiling_on_sc=False),
    )
    def body(idx_hbm, table_hbm, out_hbm, idx_v, rows_v,
             is0, is1, gs0, gs1, ss0, ss1):
        isem = (is0, is1)
        gsem = (gs0, gs1)
        ssem = (ss0, ss1)
        wid = lax.axis_index("s") * _NC + lax.axis_index("c")
        base = wid * rows_per_w

        def fire_idx_load(h, q):
            pltpu.async_copy(idx_hbm.at[pl.ds(base + h * _K, _K)],
                             idx_v.at[q], isem[q])

        def wait_idx(q):
            pltpu.make_async_copy(idx_hbm.at[pl.ds(0, _K)], idx_v.at[q],
                                  isem[q]).wait()

        def wait_gather(q, k):
            pltpu.make_async_copy(out_hbm.at[0, :, pl.ds(0, D)],
                                  rows_v.at[q, k], gsem[q]).wait()

        def wait_store(q, k):
            pltpu.make_async_copy(rows_v.at[q, k],
                                  out_hbm.at[0, :, pl.ds(0, D)],
                                  ssem[q]).wait()

        def process_half(h, q, first, prefetch):
            wait_idx(q)
            if not first:
                for k in range(_K):
                    wait_store(q, k)
            for k in range(_K):
                pltpu.async_copy(table_hbm.at[idx_v.at[q, k]],
                                 rows_v.at[q, k], gsem[q])
            if prefetch:
                fire_idx_load(h + 1, 1 - q)
            for k in range(_K):
                wait_gather(q, k)
            for k in range(_K):
                pltpu.async_copy(
                    rows_v.at[q, k],
                    out_hbm.at[base + h * _K + k, :, pl.ds(0, D)], ssem[q])

        fire_idx_load(0, 0)
        process_half(0, 0, first=True, prefetch=True)
        process_half(1, 1, first=True, prefetch=True)

        @pl.loop(2, n_halves - 2, step=2)
        def _steady(Hh):
            process_half(Hh, 0, first=False, prefetch=True)
            process_half(Hh + 1, 1, first=False, prefetch=True)

        process_half(n_halves - 2, 0, first=False, prefetch=True)
        process_half(n_halves - 1, 1, first=False, prefetch=False)

        for q in (0, 1):
            for k in range(_K):
                wait_store(q, k)

    return body


def kernel(x, table):
    B0, H = x.shape
    V, D = table.shape
    out = _emb_call(V, D, B0, H)(x.astype(jnp.int32), table)
    return out[:, :, :D]
